# baseline (speedup 1.0000x reference)
import functools

import jax
import jax.numpy as jnp
from jax import lax
from jax.experimental import pallas as pl
from jax.experimental.pallas import tpu as pltpu
from jax.experimental.pallas import tpu_sc as plsc

H, W = 512, 512
HW = H * W
B = 128
NC, NS, LANES = 2, 16, 16
NW = NC * NS
B_SC = NW
B_TC = B - B_SC
UNR = 8
NACC = 2
SPB = 8
CHUNK = 32768
NCHUNK = HW // CHUNK
UNROLL = 8
INT_MAX = 2**31 - 1


def _tc_argmax_kernel(x_ref, out_ref):
    pre = (lax.broadcasted_iota(jnp.int32, (8, W), 0) * W
           + lax.broadcasted_iota(jnp.int32, (8, W), 1))
    neg = jnp.full((8, W), -jnp.inf, jnp.float32)
    zer = jnp.zeros((8, W), jnp.int32)
    ngrp = H // 8

    for g in range(SPB):
        def body(k, carry, g=g):
            acc = list(carry)
            for t in range(UNR):
                kt = k * UNR + t
                v = x_ref[g, pl.ds(kt * 8, 8), :]
                p = t % NACC
                av, ai = acc[2 * p], acc[2 * p + 1]
                m = v > av
                acc[2 * p] = jnp.where(m, v, av)
                acc[2 * p + 1] = jnp.where(
                    m, jnp.full((8, W), kt, jnp.int32), ai)
            return tuple(acc)

        acc = list(lax.fori_loop(0, ngrp // UNR, body, (neg, zer) * NACC))
        pairs = [(acc[2 * p], acc[2 * p + 1] * (8 * W) + pre)
                 for p in range(NACC)]
        while len(pairs) > 1:
            out = []
            for q in range(0, len(pairs), 2):
                (av0, ai0), (av1, ai1) = pairs[q], pairs[q + 1]
                better = (av1 > av0) | ((av1 == av0) & (ai1 < ai0))
                out.append((jnp.where(better, av1, av0),
                            jnp.where(better, ai1, ai0)))
            pairs = out
        av, ai = pairs[0]
        m = jnp.max(av)
        cand = jnp.where(av == m, ai, jnp.int32(INT_MAX))
        out_ref[g] = jnp.broadcast_to(jnp.min(cand), (1, 128))


def _rotreduce(v, tmp, op):
    for shift in (8, 4, 2, 1):
        tmp[pl.ds(0, LANES)] = v
        tmp[pl.ds(LANES, LANES)] = v
        v = op(v, tmp[pl.ds(shift, LANES)])
    return v


def _sc_argmax_kernel(inp, outi, buf0, buf1, idxv, tmpf, tmpi,
                      sem0, sem1):
    cid = lax.axis_index("c")
    sid = lax.axis_index("s")
    wid = sid * NC + cid
    iota = lax.iota(jnp.int32, LANES)
    bufs = (buf0, buf1)
    sems = (sem0, sem1)

    def chunk_copy(c, buf, sem):
        return pltpu.make_async_copy(inp.at[wid * NCHUNK + c], buf, sem)

    chunk_copy(0, bufs[0], sems[0]).start()

    def combine(a, b):
        av, ai = a
        bv, bi = b
        better = (bv > av) | ((bv == av) & (bi < ai))
        return jnp.where(better, bv, av), jnp.where(better, bi, ai)

    neg_inf = jnp.full((LANES,), -jnp.inf, jnp.float32)
    zeros_i = jnp.zeros((LANES,), jnp.int32)
    run_max = neg_inf
    run_idx = zeros_i
    for c in range(NCHUNK):
        buf, sem = bufs[c % 2], sems[c % 2]
        if c + 1 < NCHUNK:
            chunk_copy(c + 1, bufs[(c + 1) % 2], sems[(c + 1) % 2]).start()
        chunk_copy(c, buf, sem).wait()

        def body(i, carry, buf=buf):
            i_vec = jnp.full((LANES,), i, jnp.int32)
            out = []
            for k in range(UNROLL):
                rm, ri = carry[2 * k], carry[2 * k + 1]
                v = buf[pl.ds(i * (UNROLL * LANES) + k * LANES, LANES)]
                m = v > rm
                out.append(jnp.where(m, v, rm))
                out.append(jnp.where(m, i_vec, ri))
            return tuple(out)

        init = (neg_inf, zeros_i) * UNROLL
        acc = lax.fori_loop(0, CHUNK // (UNROLL * LANES), body, init)
        pairs = []
        for k in range(UNROLL):
            rm, ri = acc[2 * k], acc[2 * k + 1]
            fi = ri * (UNROLL * LANES) + (c * CHUNK + k * LANES) + iota
            pairs.append((rm, fi))
        while len(pairs) > 1:
            pairs = [combine(pairs[j], pairs[j + 1])
                     for j in range(0, len(pairs), 2)]
        run_max, run_idx = combine((run_max, run_idx), pairs[0])

    m = _rotreduce(run_max, tmpf, jnp.maximum)
    cand = jnp.where(run_max == m, run_idx, jnp.int32(INT_MAX))
    best = _rotreduce(cand, tmpi, jnp.minimum)
    idxv[...] = best
    pltpu.sync_copy(idxv, outi.at[wid])


def _sc_gather_kernel(idx_hbm, gx, gy, outx, outy, idxv, gatv, sem):
    cid = lax.axis_index("c")
    sid = lax.axis_index("s")
    wid = sid * NC + cid

    @pl.when(wid == 0)
    def _():
        pltpu.sync_copy(idx_hbm, idxv)
        pltpu.make_async_copy(gx.at[idxv], gatv, sem).start()
        pltpu.make_async_copy(gx.at[idxv], gatv, sem).wait()
        pltpu.sync_copy(gatv, outx)
        pltpu.make_async_copy(gy.at[idxv], gatv, sem).start()
        pltpu.make_async_copy(gy.at[idxv], gatv, sem).wait()
        pltpu.sync_copy(gatv, outy)


@jax.jit
def kernel(input, grid_x, grid_y):
    xr = input.reshape(B, H, W)
    xc = input.reshape(B * NCHUNK, CHUNK)
    gx1 = grid_x.reshape(HW)
    gy1 = grid_y.reshape(HW)
    mesh = plsc.VectorSubcoreMesh(core_axis_name="c", subcore_axis_name="s")

    tc_idx3 = pl.pallas_call(
        _tc_argmax_kernel,
        out_shape=jax.ShapeDtypeStruct((B_TC, 1, 128), jnp.int32),
        grid=(B_TC // SPB,),
        in_specs=[pl.BlockSpec((SPB, H, W),
                               lambda i: (i + B_SC // SPB, 0, 0))],
        out_specs=pl.BlockSpec((SPB, 1, 128), lambda i: (i, 0, 0)),
    )(xr)

    sc_argmax = functools.partial(
        pl.kernel,
        out_type=jax.ShapeDtypeStruct((NW, LANES), jnp.int32),
        mesh=mesh,
        scratch_types=[
            pltpu.VMEM((CHUNK,), jnp.float32),
            pltpu.VMEM((CHUNK,), jnp.float32),
            pltpu.VMEM((LANES,), jnp.int32),
            pltpu.VMEM((2 * LANES,), jnp.float32),
            pltpu.VMEM((2 * LANES,), jnp.int32),
            pltpu.SemaphoreType.DMA,
            pltpu.SemaphoreType.DMA,
        ],
    )(_sc_argmax_kernel)
    sc_idx = sc_argmax(xc)

    idx_all = jnp.concatenate((sc_idx[:, 0], tc_idx3[:, 0, 0]))

    gather = functools.partial(
        pl.kernel,
        out_type=[
            jax.ShapeDtypeStruct((B,), jnp.float32),
            jax.ShapeDtypeStruct((B,), jnp.float32),
        ],
        mesh=mesh,
        scratch_types=[
            pltpu.VMEM((B,), jnp.int32),
            pltpu.VMEM((B,), jnp.float32),
            pltpu.SemaphoreType.DMA,
        ],
    )(_sc_gather_kernel)
    x, y = gather(idx_all, gx1, gy1)
    return jnp.concatenate((x.reshape(B, 1), y.reshape(B, 1)), axis=1)


# speedup vs baseline: 2.5469x; 2.5469x over previous
import functools

import jax
import jax.numpy as jnp
from jax import lax
from jax.experimental import pallas as pl
from jax.experimental.pallas import tpu as pltpu
from jax.experimental.pallas import tpu_sc as plsc

H, W = 512, 512
HW = H * W
B = 128
NC, NS, LANES = 2, 16, 16
UNR = 8
NACC = 2
SPB = 8
INT_MAX = 2**31 - 1


def _tc_argmax_kernel(x_ref, out_ref):
    pre = (lax.broadcasted_iota(jnp.int32, (8, W), 0) * W
           + lax.broadcasted_iota(jnp.int32, (8, W), 1))
    neg = jnp.full((8, W), -jnp.inf, jnp.float32)
    zer = jnp.zeros((8, W), jnp.int32)
    ngrp = H // 8

    for g in range(SPB):
        def body(k, carry, g=g):
            acc = list(carry)
            for t in range(UNR):
                kt = k * UNR + t
                v = x_ref[g, pl.ds(kt * 8, 8), :]
                p = t % NACC
                av, ai = acc[2 * p], acc[2 * p + 1]
                m = v > av
                acc[2 * p] = jnp.where(m, v, av)
                acc[2 * p + 1] = jnp.where(
                    m, jnp.full((8, W), kt, jnp.int32), ai)
            return tuple(acc)

        acc = list(lax.fori_loop(0, ngrp // UNR, body, (neg, zer) * NACC))
        pairs = [(acc[2 * p], acc[2 * p + 1] * (8 * W) + pre)
                 for p in range(NACC)]
        while len(pairs) > 1:
            out = []
            for q in range(0, len(pairs), 2):
                (av0, ai0), (av1, ai1) = pairs[q], pairs[q + 1]
                better = (av1 > av0) | ((av1 == av0) & (ai1 < ai0))
                out.append((jnp.where(better, av1, av0),
                            jnp.where(better, ai1, ai0)))
            pairs = out
        av, ai = pairs[0]
        m = jnp.max(av)
        cand = jnp.where(av == m, ai, jnp.int32(INT_MAX))
        out_ref[g] = jnp.broadcast_to(jnp.min(cand), (1, 128))


def _sc_gather_kernel(idx_hbm, gx, gy, outx, outy, idxv, gatv, sem):
    cid = lax.axis_index("c")
    sid = lax.axis_index("s")
    wid = sid * NC + cid

    @pl.when(wid == 0)
    def _():
        pltpu.sync_copy(idx_hbm, idxv)
        pltpu.make_async_copy(gx.at[idxv], gatv, sem).start()
        pltpu.make_async_copy(gx.at[idxv], gatv, sem).wait()
        pltpu.sync_copy(gatv, outx)
        pltpu.make_async_copy(gy.at[idxv], gatv, sem).start()
        pltpu.make_async_copy(gy.at[idxv], gatv, sem).wait()
        pltpu.sync_copy(gatv, outy)


@jax.jit
def kernel(input, grid_x, grid_y):
    xr = input.reshape(B, H, W)
    gx1 = grid_x.reshape(HW)
    gy1 = grid_y.reshape(HW)

    idx3 = pl.pallas_call(
        _tc_argmax_kernel,
        out_shape=jax.ShapeDtypeStruct((B, 1, 128), jnp.int32),
        grid=(B // SPB,),
        in_specs=[pl.BlockSpec((SPB, H, W), lambda i: (i, 0, 0))],
        out_specs=pl.BlockSpec((SPB, 1, 128), lambda i: (i, 0, 0)),
    )(xr)
    idx = idx3[:, 0, 0]

    gather = functools.partial(
        pl.kernel,
        out_type=[
            jax.ShapeDtypeStruct((B,), jnp.float32),
            jax.ShapeDtypeStruct((B,), jnp.float32),
        ],
        mesh=plsc.VectorSubcoreMesh(core_axis_name="c", subcore_axis_name="s"),
        scratch_types=[
            pltpu.VMEM((B,), jnp.int32),
            pltpu.VMEM((B,), jnp.float32),
            pltpu.SemaphoreType.DMA,
        ],
    )(_sc_gather_kernel)
    x, y = gather(idx, gx1, gy1)
    return jnp.concatenate((x.reshape(B, 1), y.reshape(B, 1)), axis=1)
